# baseline (device time: 145177 ns/iter reference)
import jax
import jax.numpy as jnp
from jax import lax
from jax.experimental import pallas as pl
from jax.experimental.pallas import tpu as pltpu

N_DEV = 16
P = 4
M = 2048
N = 2048
QROW = M // P
UROW = QROW // P
HALF = N // 2
C1 = 2
QC1 = HALF // C1


def kernel(A, B):
    partial = jnp.dot(
        A.astype(jnp.bfloat16),
        B.astype(jnp.bfloat16),
        preferred_element_type=jnp.float32,
    )

    def body(p_ref, out_ref, pstage_r, pstage_l, pag_r, pag_l,
             zstage_r, zstage_l, zag_r, zag_l,
             s1_ss_r, s1_rs_r, s1_ss_l, s1_rs_l,
             z_ss_r, z_rs_r, z_ss_l, z_rs_l,
             s3_ss_r, s3_rs_r, s3_ss_l, s3_rs_l):
        my = lax.axis_index("i")
        q = lax.rem(my, P)
        g = my // P
        plane_r = g * P + lax.rem(q + 1, P)
        plane_l = g * P + lax.rem(q - 1 + P, P)
        z_r = lax.rem(my + P, N_DEV)
        z_l = lax.rem(my - P + N_DEV, N_DEV)

        def mod4(k):
            return lax.rem(k + 4 * P, P)

        barrier_sem = pltpu.get_barrier_semaphore()
        for nbr in (plane_l, plane_r, z_l, z_r):
            pl.semaphore_signal(
                barrier_sem, inc=1,
                device_id=(nbr,), device_id_type=pl.DeviceIdType.MESH,
            )
        pl.semaphore_wait(barrier_sem, 4)

        pdirs = [
            dict(sgn=-1, nbr=plane_r, col=0, stage=pstage_r, ag=pag_r,
                 s1_ss=s1_ss_r, s1_rs=s1_rs_r, s3_ss=s3_ss_r, s3_rs=s3_rs_r,
                 zstage=zstage_r, zag=zag_r, z_ss=z_ss_r, z_rs=z_rs_r,
                 znbr=z_r, zsgn=-1),
            dict(sgn=+1, nbr=plane_l, col=HALF, stage=pstage_l, ag=pag_l,
                 s1_ss=s1_ss_l, s1_rs=s1_rs_l, s3_ss=s3_ss_l, s3_rs=s3_rs_l,
                 zstage=zstage_l, zag=zag_l, z_ss=z_ss_l, z_rs=z_rs_l,
                 znbr=z_l, zsgn=+1),
        ]

        def s1_send(d, j, h, slot):
            jsl = slice(j * QC1, (j + 1) * QC1)
            rdma = pltpu.make_async_remote_copy(
                src_ref=d["stage"].at[slot, :, jsl],
                dst_ref=d["ag"].at[mod4(q + d["sgn"] * h), :, jsl],
                send_sem=d["s1_ss"].at[h, j],
                recv_sem=d["s1_rs"].at[h, j],
                device_id=(d["nbr"],),
                device_id_type=pl.DeviceIdType.MESH,
            )
            rdma.start()
            return rdma

        s1 = {}
        for d in pdirs:
            d["stage"][0] = p_ref[
                pl.ds(q * QROW, QROW), d["col"]:d["col"] + HALF
            ].astype(jnp.bfloat16)
            for j in range(C1):
                s1[(d["sgn"], j)] = [s1_send(d, j, 0, 0)]

        for h in range(P - 1):
            for j in range(C1):
                for d in pdirs:
                    cj = d["col"] + j * QC1
                    t_r = mod4(q + d["sgn"] * (h + 1))
                    rd = s1[(d["sgn"], j)]
                    rd[h].wait_recv()
                    tmp = (
                        p_ref[pl.ds(t_r * QROW, QROW), cj:cj + QC1]
                        + d["ag"][t_r, :, j * QC1:(j + 1) * QC1].astype(
                            jnp.float32)
                    )
                    if h < P - 2:
                        slot = (h + 1) % 2
                        if h >= 1:
                            rd[h - 1].wait_send()
                        d["stage"][slot, :, j * QC1:(j + 1) * QC1] = (
                            tmp.astype(jnp.bfloat16))
                        rd.append(s1_send(d, j, h + 1, slot))
                    else:
                        out_ref[pl.ds(t_r * QROW, QROW), cj:cj + QC1] = tmp
        for rd in s1.values():
            rd[P - 3].wait_send()
            rd[P - 2].wait_send()

        def z_send(d, h, sem_h, src_buf, src_slot, rb):
            rdma = pltpu.make_async_remote_copy(
                src_ref=src_buf.at[src_slot],
                dst_ref=d["zag"].at[mod4(g + d["zsgn"] * h)],
                send_sem=d["z_ss"].at[sem_h],
                recv_sem=d["z_rs"].at[sem_h],
                device_id=(d["znbr"],),
                device_id_type=pl.DeviceIdType.MESH,
            )
            rdma.start()
            return rdma

        z = {}
        for d in pdirs:
            d["rb"] = mod4(q - d["sgn"]) * QROW
            d["zstage"][0] = out_ref[
                pl.ds(d["rb"] + g * UROW, UROW), d["col"]:d["col"] + HALF
            ].astype(jnp.bfloat16)
            z[d["sgn"]] = [z_send(d, 0, 0, d["zstage"], 0, d["rb"])]

        for h in range(P - 1):
            for d in pdirs:
                u_r = mod4(g + d["zsgn"] * (h + 1))
                rd = z[d["sgn"]]
                rd[h].wait_recv()
                tmp = (
                    out_ref[pl.ds(d["rb"] + u_r * UROW, UROW),
                            d["col"]:d["col"] + HALF]
                    + d["zag"][u_r].astype(jnp.float32)
                )
                if h < P - 2:
                    slot = (h + 1) % 2
                    if h >= 1:
                        rd[h - 1].wait_send()
                    d["zstage"][slot] = tmp.astype(jnp.bfloat16)
                    rd.append(z_send(d, h + 1, h + 1, d["zstage"], slot,
                                     d["rb"]))
                else:
                    out_ref[pl.ds(d["rb"] + u_r * UROW, UROW),
                            d["col"]:d["col"] + HALF] = tmp
        for rd in z.values():
            rd[P - 3].wait_send()
            rd[P - 2].wait_send()

        def zag_send(d, h):
            u_s = mod4(g - d["zsgn"] + d["zsgn"] * h)
            rdma = pltpu.make_async_remote_copy(
                src_ref=d["zag"].at[u_s],
                dst_ref=d["zag"].at[u_s],
                send_sem=d["z_ss"].at[P - 1 + h],
                recv_sem=d["z_rs"].at[P - 1 + h],
                device_id=(d["znbr"],),
                device_id_type=pl.DeviceIdType.MESH,
            )
            rdma.start()
            return rdma

        zag = {}
        for d in pdirs:
            u_own = mod4(g - d["zsgn"])
            d["zag"][u_own] = out_ref[
                pl.ds(d["rb"] + u_own * UROW, UROW), d["col"]:d["col"] + HALF
            ].astype(jnp.bfloat16)
            zag[d["sgn"]] = [zag_send(d, 0)]
        for h in range(P - 1):
            for d in pdirs:
                u_g = mod4(g + d["zsgn"] * h)
                rd = zag[d["sgn"]]
                rd[h].wait_recv()
                if h < P - 2:
                    rd.append(zag_send(d, h + 1))
                out_ref[pl.ds(d["rb"] + u_g * UROW, UROW),
                        d["col"]:d["col"] + HALF] = (
                    d["zag"][u_g].astype(jnp.float32))
        for rd in zag.values():
            for rdma in rd:
                rdma.wait_send()

        def s3_send(d, j, h):
            jsl = slice(j * QC1, (j + 1) * QC1)
            t_s = mod4(q - d["sgn"] + d["sgn"] * h)
            rdma = pltpu.make_async_remote_copy(
                src_ref=d["ag"].at[t_s, :, jsl],
                dst_ref=d["ag"].at[t_s, :, jsl],
                send_sem=d["s3_ss"].at[h, j],
                recv_sem=d["s3_rs"].at[h, j],
                device_id=(d["nbr"],),
                device_id_type=pl.DeviceIdType.MESH,
            )
            rdma.start()
            return rdma

        s3 = {}
        for d in pdirs:
            t_own = mod4(q - d["sgn"])
            d["ag"][t_own] = out_ref[
                pl.ds(t_own * QROW, QROW), d["col"]:d["col"] + HALF
            ].astype(jnp.bfloat16)
            for j in range(C1):
                s3[(d["sgn"], j)] = [s3_send(d, j, 0)]
        for h in range(P - 1):
            for j in range(C1):
                for d in pdirs:
                    cj = d["col"] + j * QC1
                    t_g = mod4(q + d["sgn"] * h)
                    rd = s3[(d["sgn"], j)]
                    rd[h].wait_recv()
                    if h < P - 2:
                        rd.append(s3_send(d, j, h + 1))
                    out_ref[pl.ds(t_g * QROW, QROW), cj:cj + QC1] = (
                        d["ag"][t_g, :, j * QC1:(j + 1) * QC1].astype(
                            jnp.float32))
        for rd in s3.values():
            for rdma in rd:
                rdma.wait_send()

    return pl.pallas_call(
        body,
        out_shape=jax.ShapeDtypeStruct((M, N), jnp.float32),
        in_specs=[pl.BlockSpec(memory_space=pltpu.VMEM)],
        out_specs=pl.BlockSpec(memory_space=pltpu.VMEM),
        scratch_shapes=[
            pltpu.VMEM((2, QROW, HALF), jnp.bfloat16),
            pltpu.VMEM((2, QROW, HALF), jnp.bfloat16),
            pltpu.VMEM((P, QROW, HALF), jnp.bfloat16),
            pltpu.VMEM((P, QROW, HALF), jnp.bfloat16),
            pltpu.VMEM((2, UROW, HALF), jnp.bfloat16),
            pltpu.VMEM((2, UROW, HALF), jnp.bfloat16),
            pltpu.VMEM((P, UROW, HALF), jnp.bfloat16),
            pltpu.VMEM((P, UROW, HALF), jnp.bfloat16),
            pltpu.SemaphoreType.DMA((P - 1, C1)),
            pltpu.SemaphoreType.DMA((P - 1, C1)),
            pltpu.SemaphoreType.DMA((P - 1, C1)),
            pltpu.SemaphoreType.DMA((P - 1, C1)),
            pltpu.SemaphoreType.DMA((2 * (P - 1),)),
            pltpu.SemaphoreType.DMA((2 * (P - 1),)),
            pltpu.SemaphoreType.DMA((2 * (P - 1),)),
            pltpu.SemaphoreType.DMA((2 * (P - 1),)),
            pltpu.SemaphoreType.DMA((P - 1, C1)),
            pltpu.SemaphoreType.DMA((P - 1, C1)),
            pltpu.SemaphoreType.DMA((P - 1, C1)),
            pltpu.SemaphoreType.DMA((P - 1, C1)),
        ],
        compiler_params=pltpu.CompilerParams(collective_id=0),
    )(partial)
